# baseline (speedup 1.0000x reference)
import functools
import math

import jax
import jax.numpy as jnp
from jax.experimental import pallas as pl
from jax.experimental.pallas import tpu as pltpu
from jax.experimental.pallas import tpu_sc as plsc

Q_NUM = 50
MAX_CHOICES = 8
HEADS = 8
N_ANN = 100
EMB_DIM = 128
SENT_DIM = 384
FEAT_DIM = EMB_DIM + MAX_CHOICES + SENT_DIM
D_FF = 4 * FEAT_DIM
B, S = 16, 512
DH = FEAT_DIM // HEADS
HP = 128
HDIM = HEADS * HP
HALF = FEAT_DIM // 2
F32 = jnp.float32


BF16 = jnp.bfloat16


def _dot(a, b, dims):
    return jax.lax.dot_general(a.astype(BF16), b.astype(BF16),
                               (dims, ((), ())),
                               preferred_element_type=F32,
                               precision=jax.lax.Precision.DEFAULT)


def _ln(z, a, b):
    m = jnp.mean(z, axis=1, keepdims=True)
    d = z - m
    var = jnp.sum(d * d, axis=1, keepdims=True) / (FEAT_DIM - 1)
    return a * d / (jnp.sqrt(var) + 1e-6) + b


_NC, _NS = 2, 16
_NW = _NC * _NS
_BS = B * S
_PER_W = _BS // _NW


def _sc_gather_call(qidx, aidx, qtab, atab):
    mesh = plsc.VectorSubcoreMesh(core_axis_name="c", subcore_axis_name="s")

    @functools.partial(
        pl.kernel,
        mesh=mesh,
        out_type=[jax.ShapeDtypeStruct((_BS, EMB_DIM), F32)],
        scratch_types=[
            pltpu.VMEM((_PER_W,), jnp.int32),
            pltpu.VMEM((_PER_W,), jnp.int32),
            pltpu.VMEM((_PER_W, EMB_DIM), F32),
            pltpu.VMEM((_PER_W, EMB_DIM), F32),
            pltpu.SemaphoreType.DMA,
            pltpu.SemaphoreType.DMA,
        ],
    )
    def k(qidx_hbm, aidx_hbm, qtab_hbm, atab_hbm, out_hbm,
          qidx_v, aidx_v, qrows_v, arows_v, qsem, asem):
        wid = jax.lax.axis_index("s") * _NC + jax.lax.axis_index("c")
        base = wid * _PER_W
        pltpu.sync_copy(qidx_hbm.at[pl.ds(base, _PER_W)], qidx_v)
        pltpu.sync_copy(aidx_hbm.at[pl.ds(base, _PER_W)], aidx_v)
        cq = pltpu.async_copy(qtab_hbm.at[qidx_v], qrows_v, qsem)
        ca = pltpu.async_copy(atab_hbm.at[aidx_v], arows_v, asem)
        cq.wait()
        ca.wait()

        nchunk = EMB_DIM // 16

        def add_chunk(i, carry):
            r = i // nchunk
            c = (i % nchunk) * 16
            qrows_v[r, pl.ds(c, 16)] = (qrows_v[r, pl.ds(c, 16)] +
                                        arows_v[r, pl.ds(c, 16)])
            return carry

        jax.lax.fori_loop(0, _PER_W * nchunk, add_chunk, 0)
        pltpu.sync_copy(qrows_v, out_hbm.at[pl.ds(base, _PER_W)])

    return k(qidx, aidx, qtab, atab)[0]


def _layer_body(fx_ref, px_ref, qcol_ref, qrow_ref,
                wqkv_ref, bqkv_ref,
                ow_ref, ob_ref, n1a_ref, n1b_ref,
                f1w_ref, f1b_ref, f2w_ref, f2b_ref, n2a_ref, n2b_ref,
                puf_ref, pup_ref, pub_ref,
                s1w_ref, s1b_ref, sc2w_ref, sc2b_ref,
                c1w_ref, c1b_ref,
                fxo_ref, pxo_ref):
    fx = fx_ref[...]
    px = px_ref[...]

    qkv = _dot(fx, wqkv_ref[...], ((1,), (1,))) + bqkv_ref[0:1, :]
    q = qkv[:, :HDIM]
    k = qkv[:, HDIM:2 * HDIM]
    v = qkv[:, 2 * HDIM:]
    scale = 1.0 / math.sqrt(DH)
    outs = []
    for h in range(HEADS):
        sl = slice(h * HP, (h + 1) * HP)
        sc = _dot(q[:, sl], k[:, sl], ((1,), (1,))) * scale
        e = jnp.exp(sc)
        attn = e / jnp.sum(e, axis=1, keepdims=True)
        outs.append(_dot(attn, v[:, sl], ((1,), (0,))))
    out_all = jnp.concatenate(outs, axis=1)
    attn_out = _dot(out_all, ow_ref[...], ((1,), (1,))) + ob_ref[0:1, :]

    fx1 = _ln(fx + attn_out, n1a_ref[0:1, :], n1b_ref[0:1, :])
    ffh = jnp.maximum(_dot(fx1, f1w_ref[...], ((1,), (1,))) + f1b_ref[0:1, :],
                      0.0)
    ff = _dot(ffh, f2w_ref[...], ((1,), (1,))) + f2b_ref[0:1, :]
    fx2 = _ln(fx1 + ff, n2a_ref[0:1, :], n2b_ref[0:1, :])
    fxo_ref[...] = fx2

    pxl = (_dot(fx2, puf_ref[...], ((1,), (1,))) +
           _dot(px, pup_ref[...], ((1,), (1,))) + pub_ref[0:1, :])

    simh = jnp.maximum(_dot(fx2, s1w_ref[...], ((1,), (1,))) + s1b_ref[0:1, :],
                       0.0)
    confh = jnp.maximum(_dot(fx2, c1w_ref[...], ((1,), (1,))) +
                        c1b_ref[0:1, :], 0.0)
    sch = jnp.concatenate([simh, confh], axis=1)
    sc2 = _dot(sch, sc2w_ref[...], ((1,), (0,))) + sc2b_ref[0:1, :]
    sim = sc2[:, 0:1]
    conf = jax.nn.sigmoid(sc2[:, 1:2])

    qc = qcol_ref[:, 0:1]
    qr = qrow_ref[0:1, :]
    mask = (qc == qr).astype(F32)
    sm = sim * mask
    e = jnp.exp(sm)
    aw = e / jnp.sum(e, axis=0, keepdims=True)
    smoothed = _dot(aw, pxl, ((0,), (0,)))
    pxo_ref[...] = conf * pxl + (1.0 - conf) * smoothed


def _full(shape):
    return pl.BlockSpec(shape, lambda b: (0,) * len(shape))


def _vec8(v):
    v = jnp.asarray(v, F32).reshape(1, -1)
    return jnp.broadcast_to(v, (8, v.shape[1]))


def _prep_layer(p):
    pad = HP - DH
    wqkv = jnp.pad(
        jnp.stack([p['Qw'], p['Kw'], p['Vw']]).astype(BF16)
        .reshape(3 * HEADS, DH, FEAT_DIM),
        ((0, 0), (0, pad), (0, 0))).reshape(3 * HDIM, FEAT_DIM)
    bqkv = _vec8(jnp.pad(
        jnp.stack([p['Qb'], p['Kb'], p['Vb']]).reshape(3 * HEADS, DH),
        ((0, 0), (0, pad))).reshape(3 * HDIM))
    ow = jnp.pad(p['Ow'].astype(BF16).reshape(FEAT_DIM, HEADS, DH),
                 ((0, 0), (0, 0), (0, pad))).reshape(FEAT_DIM, HDIM)
    return (wqkv, bqkv,
            ow, _vec8(p['Ob']), _vec8(p['n1a']), _vec8(p['n1b']),
            p['ff1w'].astype(BF16), _vec8(p['ff1b']),
            p['ff2w'].astype(BF16), _vec8(p['ff2b']),
            _vec8(p['n2a']), _vec8(p['n2b']),
            p['puw'][:, :FEAT_DIM].astype(BF16),
            p['puw'][:, FEAT_DIM:].astype(BF16), _vec8(p['pub']),
            p['sim1w'].astype(BF16), _vec8(p['sim1b']),
            _sc2w(p), _vec8(_sc2b(p)),
            p['conf1w'].astype(BF16), _vec8(p['conf1b']))


def _sc2w(p):
    z = jnp.zeros((HALF,), F32)
    col0 = jnp.concatenate([p['sim2w'][0], z])
    col1 = jnp.concatenate([z, p['conf2w'][0]])
    w = jnp.stack([col0, col1], axis=1)
    return jnp.pad(w, ((0, 0), (0, 6))).astype(BF16)


def _sc2b(p):
    return jnp.pad(jnp.stack([p['sim2b'][0], p['conf2b'][0]]), (0, 6))


def _layer_call(fx, px, qcol, qrow, wts):
    in_specs = [
        pl.BlockSpec((None, S, FEAT_DIM), lambda b: (b, 0, 0)),
        pl.BlockSpec((None, S, MAX_CHOICES), lambda b: (b, 0, 0)),
        pl.BlockSpec((None, S, 8), lambda b: (b, 0, 0)),
        pl.BlockSpec((None, 8, S), lambda b: (b, 0, 0)),
    ] + [_full(w.shape) for w in wts]
    return pl.pallas_call(
        _layer_body,
        grid=(B,),
        in_specs=in_specs,
        out_specs=[
            pl.BlockSpec((None, S, FEAT_DIM), lambda b: (b, 0, 0)),
            pl.BlockSpec((None, S, MAX_CHOICES), lambda b: (b, 0, 0)),
        ],
        out_shape=[
            jax.ShapeDtypeStruct((B, S, FEAT_DIM), F32),
            jax.ShapeDtypeStruct((B, S, MAX_CHOICES), F32),
        ],
        compiler_params=pltpu.CompilerParams(
            dimension_semantics=("arbitrary",)),
    )(fx, px, qcol, qrow, *wts)


def kernel(x, annotators, questions, embeddings, annotator_embedding,
           question_embedding, layer_params):
    qf = questions.astype(F32)
    qcol = jnp.broadcast_to(qf[:, :, None], (B, S, 8))
    qrow = jnp.broadcast_to(qf[:, None, :], (B, 8, S))
    qidx = questions.reshape(_BS)
    aidx = jnp.where(annotators < 0, N_ANN, annotators).reshape(_BS)
    emb_sum = _sc_gather_call(qidx, aidx, question_embedding,
                              annotator_embedding).reshape(B, S, EMB_DIM)
    fx = jnp.concatenate([emb_sum, embeddings, x[:, :, 1:]], axis=-1)
    px = x[:, :, 1:]
    for p in layer_params:
        fx, px = _layer_call(fx, px, qcol, qrow, _prep_layer(p))
    return px


# speedup vs baseline: 1.0042x; 1.0042x over previous
import functools
import math

import jax
import jax.numpy as jnp
from jax.experimental import pallas as pl
from jax.experimental.pallas import tpu as pltpu
from jax.experimental.pallas import tpu_sc as plsc

Q_NUM = 50
MAX_CHOICES = 8
HEADS = 8
N_ANN = 100
EMB_DIM = 128
SENT_DIM = 384
FEAT_DIM = EMB_DIM + MAX_CHOICES + SENT_DIM
D_FF = 4 * FEAT_DIM
B, S = 16, 512
DH = FEAT_DIM // HEADS
HP = 128
HDIM = HEADS * HP
HALF = FEAT_DIM // 2
F32 = jnp.float32


BF16 = jnp.bfloat16


def _dot(a, b, dims):
    return jax.lax.dot_general(a.astype(BF16), b.astype(BF16),
                               (dims, ((), ())),
                               preferred_element_type=F32,
                               precision=jax.lax.Precision.DEFAULT)


def _ln(z, a, b):
    m = jnp.mean(z, axis=1, keepdims=True)
    d = z - m
    var = jnp.sum(d * d, axis=1, keepdims=True) / (FEAT_DIM - 1)
    return a * d / (jnp.sqrt(var) + 1e-6) + b


_NC, _NS = 2, 16
_NW = _NC * _NS
_BS = B * S
_PER_W = _BS // _NW


def _sc_gather_call(qidx, aidx, qtab, atab):
    mesh = plsc.VectorSubcoreMesh(core_axis_name="c", subcore_axis_name="s")

    @functools.partial(
        pl.kernel,
        mesh=mesh,
        out_type=[jax.ShapeDtypeStruct((_BS, EMB_DIM), F32)],
        scratch_types=[
            pltpu.VMEM((_PER_W,), jnp.int32),
            pltpu.VMEM((_PER_W,), jnp.int32),
            pltpu.VMEM((_PER_W, EMB_DIM), F32),
            pltpu.VMEM((_PER_W, EMB_DIM), F32),
            pltpu.SemaphoreType.DMA,
            pltpu.SemaphoreType.DMA,
        ],
    )
    def k(qidx_hbm, aidx_hbm, qtab_hbm, atab_hbm, out_hbm,
          qidx_v, aidx_v, qrows_v, arows_v, qsem, asem):
        wid = jax.lax.axis_index("s") * _NC + jax.lax.axis_index("c")
        base = wid * _PER_W
        pltpu.sync_copy(qidx_hbm.at[pl.ds(base, _PER_W)], qidx_v)
        pltpu.sync_copy(aidx_hbm.at[pl.ds(base, _PER_W)], aidx_v)
        cq = pltpu.async_copy(qtab_hbm.at[qidx_v], qrows_v, qsem)
        ca = pltpu.async_copy(atab_hbm.at[aidx_v], arows_v, asem)
        cq.wait()
        ca.wait()

        nchunk = EMB_DIM // 16

        def add_chunk(i, carry):
            r = i // nchunk
            c = (i % nchunk) * 16
            qrows_v[r, pl.ds(c, 16)] = (qrows_v[r, pl.ds(c, 16)] +
                                        arows_v[r, pl.ds(c, 16)])
            return carry

        jax.lax.fori_loop(0, _PER_W * nchunk, add_chunk, 0)
        pltpu.sync_copy(qrows_v, out_hbm.at[pl.ds(base, _PER_W)])

    return k(qidx, aidx, qtab, atab)[0]


def _layer_body(fx_ref, px_ref, qcol_ref, qrow_ref,
                wqkv_ref, bqkv_ref,
                ow_ref, ob_ref, n1a_ref, n1b_ref,
                f1w_ref, f1b_ref, f2w_ref, f2b_ref, n2a_ref, n2b_ref,
                puf_ref, pup_ref, pub_ref,
                s1w_ref, s1b_ref, s2w_ref, s2b_ref,
                c1w_ref, c1b_ref, c2w_ref, c2b_ref,
                fxo_ref, pxo_ref):
    fx = fx_ref[...]
    px = px_ref[...]

    qkv = _dot(fx, wqkv_ref[...], ((1,), (1,))) + bqkv_ref[0:1, :]
    q = qkv[:, :HDIM]
    k = qkv[:, HDIM:2 * HDIM]
    v = qkv[:, 2 * HDIM:]
    scale = 1.0 / math.sqrt(DH)
    outs = []
    for h in range(HEADS):
        sl = slice(h * HP, (h + 1) * HP)
        sc = _dot(q[:, sl], k[:, sl], ((1,), (1,))) * scale
        e = jnp.exp(sc)
        attn = e / jnp.sum(e, axis=1, keepdims=True)
        outs.append(_dot(attn, v[:, sl], ((1,), (0,))))
    out_all = jnp.concatenate(outs, axis=1)
    attn_out = _dot(out_all, ow_ref[...], ((1,), (1,))) + ob_ref[0:1, :]

    fx1 = _ln(fx + attn_out, n1a_ref[0:1, :], n1b_ref[0:1, :])
    ffh = jnp.maximum(_dot(fx1, f1w_ref[...], ((1,), (1,))) + f1b_ref[0:1, :],
                      0.0)
    ff = _dot(ffh, f2w_ref[...], ((1,), (1,))) + f2b_ref[0:1, :]
    fx2 = _ln(fx1 + ff, n2a_ref[0:1, :], n2b_ref[0:1, :])
    fxo_ref[...] = fx2

    pxl = (_dot(fx2, puf_ref[...], ((1,), (1,))) +
           _dot(px, pup_ref[...], ((1,), (1,))) + pub_ref[0:1, :])

    simh = jnp.maximum(_dot(fx2, s1w_ref[...], ((1,), (1,))) + s1b_ref[0:1, :],
                       0.0)
    sim = (jnp.sum(simh * s2w_ref[0:1, :], axis=1, keepdims=True) +
           s2b_ref[0:1, 0:1])
    confh = jnp.maximum(_dot(fx2, c1w_ref[...], ((1,), (1,))) +
                        c1b_ref[0:1, :], 0.0)
    conf = jax.nn.sigmoid(
        jnp.sum(confh * c2w_ref[0:1, :], axis=1, keepdims=True) +
        c2b_ref[0:1, 0:1])

    qc = qcol_ref[:, 0:1]
    qr = qrow_ref[0:1, :]
    mask = (qc == qr).astype(F32)
    sm = sim * mask
    e = jnp.exp(sm)
    aw = e / jnp.sum(e, axis=0, keepdims=True)
    smoothed = _dot(aw, pxl, ((0,), (0,)))
    pxo_ref[...] = conf * pxl + (1.0 - conf) * smoothed


def _full(shape):
    return pl.BlockSpec(shape, lambda b: (0,) * len(shape))


def _vec8(v):
    v = jnp.asarray(v, F32).reshape(1, -1)
    return jnp.broadcast_to(v, (8, v.shape[1]))


def _prep_layer(p):
    pad = HP - DH
    wqkv = jnp.pad(
        jnp.stack([p['Qw'], p['Kw'], p['Vw']]).astype(BF16)
        .reshape(3 * HEADS, DH, FEAT_DIM),
        ((0, 0), (0, pad), (0, 0))).reshape(3 * HDIM, FEAT_DIM)
    bqkv = _vec8(jnp.pad(
        jnp.stack([p['Qb'], p['Kb'], p['Vb']]).reshape(3 * HEADS, DH),
        ((0, 0), (0, pad))).reshape(3 * HDIM))
    ow = jnp.pad(p['Ow'].astype(BF16).reshape(FEAT_DIM, HEADS, DH),
                 ((0, 0), (0, 0), (0, pad))).reshape(FEAT_DIM, HDIM)
    return (wqkv, bqkv,
            ow, _vec8(p['Ob']), _vec8(p['n1a']), _vec8(p['n1b']),
            p['ff1w'].astype(BF16), _vec8(p['ff1b']),
            p['ff2w'].astype(BF16), _vec8(p['ff2b']),
            _vec8(p['n2a']), _vec8(p['n2b']),
            p['puw'][:, :FEAT_DIM].astype(BF16),
            p['puw'][:, FEAT_DIM:].astype(BF16), _vec8(p['pub']),
            p['sim1w'].astype(BF16), _vec8(p['sim1b']), _vec8(p['sim2w'][0]),
            _vec8(jnp.broadcast_to(p['sim2b'], (8,))),
            p['conf1w'].astype(BF16), _vec8(p['conf1b']), _vec8(p['conf2w'][0]),
            _vec8(jnp.broadcast_to(p['conf2b'], (8,))))


def _layer_call(fx, px, qcol, qrow, wts):
    in_specs = [
        pl.BlockSpec((None, S, FEAT_DIM), lambda b: (b, 0, 0)),
        pl.BlockSpec((None, S, MAX_CHOICES), lambda b: (b, 0, 0)),
        pl.BlockSpec((None, S, 8), lambda b: (b, 0, 0)),
        pl.BlockSpec((None, 8, S), lambda b: (b, 0, 0)),
    ] + [_full(w.shape) for w in wts]
    return pl.pallas_call(
        _layer_body,
        grid=(B,),
        in_specs=in_specs,
        out_specs=[
            pl.BlockSpec((None, S, FEAT_DIM), lambda b: (b, 0, 0)),
            pl.BlockSpec((None, S, MAX_CHOICES), lambda b: (b, 0, 0)),
        ],
        out_shape=[
            jax.ShapeDtypeStruct((B, S, FEAT_DIM), F32),
            jax.ShapeDtypeStruct((B, S, MAX_CHOICES), F32),
        ],
        compiler_params=pltpu.CompilerParams(
            dimension_semantics=("arbitrary",)),
    )(fx, px, qcol, qrow, *wts)


def kernel(x, annotators, questions, embeddings, annotator_embedding,
           question_embedding, layer_params):
    qf = questions.astype(F32)
    qcol = jnp.broadcast_to(qf[:, :, None], (B, S, 8))
    qrow = jnp.broadcast_to(qf[:, None, :], (B, 8, S))
    qidx = questions.reshape(_BS)
    aidx = jnp.where(annotators < 0, N_ANN, annotators).reshape(_BS)
    emb_sum = _sc_gather_call(qidx, aidx, question_embedding,
                              annotator_embedding).reshape(B, S, EMB_DIM)
    fx = jnp.concatenate([emb_sum, embeddings, x[:, :, 1:]], axis=-1)
    px = x[:, :, 1:]
    for p in layer_params:
        fx, px = _layer_call(fx, px, qcol, qrow, _prep_layer(p))
    return px
